# baseline (speedup 1.0000x reference)
import functools
import jax
import jax.numpy as jnp
from jax import lax
from jax.experimental import pallas as pl
from jax.experimental.pallas import tpu as pltpu
from jax.experimental.pallas import tpu_sc as plsc

_B, _C, _H, _W = 16, 256, 32, 32
_L = 16


def _sc_body(row_hbm, col_hbm, out_hbm, buf, colbuf, tmp, sem):
    cid = lax.axis_index("c")
    sid = lax.axis_index("s")
    y = sid * 2 + cid
    pltpu.sync_copy(col_hbm.at[pl.ds(0, _W)], colbuf)
    pltpu.sync_copy(row_hbm.at[y], tmp)
    for i in range(_C // _L):
        rv = tmp[pl.ds(i * _L, _L)]
        for x in range(_W):
            buf[x, pl.ds(_C + i * _L, _L)] = rv
            buf[x, pl.ds(i * _L, _L)] = colbuf[x, pl.ds(i * _L, _L)]
    for b in range(_B):
        pltpu.make_async_copy(buf, out_hbm.at[b, y], sem).start()
    for b in range(_B):
        pltpu.make_async_copy(buf, out_hbm.at[b, y], sem).wait()


def kernel(mask, row_embed, col_embed):
    b = mask.shape[0]
    h, w = mask.shape[-2], mask.shape[-1]
    c = row_embed.shape[-1]
    mesh = plsc.VectorSubcoreMesh(core_axis_name="c", subcore_axis_name="s")
    k = functools.partial(
        pl.kernel,
        mesh=mesh,
        out_type=jax.ShapeDtypeStruct((b, h, w, 2 * c), jnp.float32),
        scratch_types=[
            pltpu.VMEM((w, 2 * c), jnp.float32),
            pltpu.VMEM((w, c), jnp.float32),
            pltpu.VMEM((c,), jnp.float32),
            pltpu.SemaphoreType.DMA,
        ],
    )(_sc_body)
    out = k(row_embed, col_embed)
    return jnp.transpose(out, (0, 3, 1, 2))


# speedup vs baseline: 2.8498x; 2.8498x over previous
import jax
import jax.numpy as jnp
from jax.experimental import pallas as pl
from jax.experimental.pallas import tpu as pltpu

_B, _C, _H, _W = 16, 256, 32, 32


_SPLIT = 1


def _body(row_ref, col_ref, out_ref, scratch, sems):
    scratch[:, :, :_C] = jnp.broadcast_to(col_ref[...][None, :, :], (_H, _W, _C))
    scratch[:, :, _C:] = jnp.broadcast_to(row_ref[...][:, None, :], (_H, _W, _C))
    hh = _H // _SPLIT
    for b in range(_B):
        for s in range(_SPLIT):
            pltpu.make_async_copy(
                scratch.at[pl.ds(s * hh, hh)],
                out_ref.at[b, pl.ds(s * hh, hh)],
                sems.at[b * _SPLIT + s]).start()
    for b in range(_B):
        for s in range(_SPLIT):
            pltpu.make_async_copy(
                scratch.at[pl.ds(s * hh, hh)],
                out_ref.at[b, pl.ds(s * hh, hh)],
                sems.at[b * _SPLIT + s]).wait()


def kernel(mask, row_embed, col_embed):
    b = mask.shape[0]
    h, w = mask.shape[-2], mask.shape[-1]
    c = row_embed.shape[-1]
    out = pl.pallas_call(
        _body,
        grid=(1,),
        in_specs=[
            pl.BlockSpec((h, c), lambda i: (0, 0)),
            pl.BlockSpec((w, c), lambda i: (0, 0)),
        ],
        out_specs=pl.BlockSpec(memory_space=pl.ANY),
        out_shape=jax.ShapeDtypeStruct((b, h, w, 2 * c), jnp.float32),
        scratch_shapes=[
            pltpu.VMEM((h, w, 2 * c), jnp.float32),
            pltpu.SemaphoreType.DMA((b * _SPLIT,)),
        ],
    )(row_embed, col_embed)
    return jnp.transpose(out, (0, 3, 1, 2))
